# baseline (speedup 1.0000x reference)
import functools

import jax
import jax.numpy as jnp
import numpy as np
from jax.experimental import pallas as pl
from jax.experimental.pallas import tpu as pltpu

F32 = jnp.float32


def _conv2d_toeplitz(wk, hh, ww, pad, dil):
    co, ci, kh, kw = wk.shape
    io = np.arange(hh).reshape(hh, 1, 1, 1, 1, 1)
    jo = np.arange(ww).reshape(1, ww, 1, 1, 1, 1)
    ii = np.arange(hh).reshape(1, 1, hh, 1, 1, 1)
    jj = np.arange(ww).reshape(1, 1, 1, ww, 1, 1)
    ka = np.arange(kh).reshape(1, 1, 1, 1, kh, 1)
    kb = np.arange(kw).reshape(1, 1, 1, 1, 1, kw)
    sel = ((ii == io + ka * dil - pad) & (jj == jo + kb * dil - pad)).astype(np.float32)
    m = jnp.einsum("pqijab,ocab->opqcij", jnp.asarray(sel), wk,
                   precision=jax.lax.Precision.HIGHEST)
    return m.reshape(co * hh * ww, ci * hh * ww)


def _bn_affine(rowsum, rowssq, count, gamma, beta, eps):
    mean = rowsum / count
    var = rowssq / count - mean * mean
    inv_std = jax.lax.rsqrt(var + eps)
    scale = gamma * inv_std
    shift = beta - mean * scale
    return scale, shift


def _pass1_kernel(x_ref, w_ref, y_ref, sum_ref, ssq_ref, *, bb):
    @pl.when(pl.program_id(1) == 0)
    def _():
        sum_ref[...] = jnp.zeros_like(sum_ref)
        ssq_ref[...] = jnp.zeros_like(ssq_ref)

    wmat = w_ref[...]
    s_acc = jnp.zeros((512, 1), F32)
    q_acc = jnp.zeros((512, 1), F32)
    for bloc in range(bb):
        piece = x_ref[bloc]
        xp = jnp.concatenate(
            [piece[:, vv * 64:(vv + 1) * 64] for vv in range(8)], axis=0)
        y = jnp.dot(wmat, xp, preferred_element_type=F32)
        s_acc += jnp.sum(y, axis=1, keepdims=True)
        q_acc += jnp.sum(y * y, axis=1, keepdims=True)
        t = jnp.swapaxes(y.reshape(8, 64, 64), 1, 2).reshape(512, 64)
        y_ref[:, bloc * 64:(bloc + 1) * 64] = t.astype(jnp.bfloat16)
    sum_ref[0] += s_acc
    ssq_ref[0] += q_acc


def _pass2_kernel(x_ref, w_ref, sc_ref, sh_ref, y_ref, sum_ref, ssq_ref,
                  gram_ref):
    x = jnp.maximum(x_ref[...].astype(jnp.float32) * sc_ref[...]
                    + sh_ref[...], 0.0)
    y = jnp.dot(w_ref[...], x, preferred_element_type=F32)
    y_ref[...] = y.astype(jnp.bfloat16)

    @pl.when(pl.program_id(1) == 0)
    def _():
        sum_ref[...] = jnp.zeros_like(sum_ref)
        ssq_ref[...] = jnp.zeros_like(ssq_ref)
        gram_ref[...] = jnp.zeros_like(gram_ref)

    sum_ref[0] += jnp.sum(y, axis=1, keepdims=True)
    ssq_ref[0] += jnp.sum(y * y, axis=1, keepdims=True)
    g = jax.lax.dot_general(y, y, (((1,), (1,)), ((), ())),
                            preferred_element_type=F32)
    gram_ref[0] += g


def _pass3_kernel(x_ref, w_ref, c_ref, o_ref, *, bb):
    flats = []
    for bloc in range(bb):
        piece = x_ref[:, bloc * 64:(bloc + 1) * 64]
        sw = jnp.swapaxes(piece.reshape(8, 64, 64), 1, 2)
        flats.append(sw.reshape(8, 4096))
    t = jnp.concatenate(flats, axis=0).astype(jnp.float32)
    o_ref[...] = (jnp.dot(w_ref[...], t, preferred_element_type=F32)
                  + c_ref[...])


def kernel(x, w_conv2, g2, b2, w_conv1, g1, b1, w_proj, gp, bp):
    eps = 1e-5
    b, c, u, v, h, w = x.shape
    assert (c, u, v, h, w) == (8, 8, 8, 8, 8) and b % 32 == 0
    x = x.astype(F32)
    n_a = b * h * w
    n_c = n_a * 64

    cores = 2
    bb1 = 16
    tn = bb1 * 64
    nt = n_a // (cores * tn)

    wa_big = _conv2d_toeplitz(w_conv2.astype(F32), u, v, pad=1, dil=1)
    perm = np.empty(512, dtype=np.int32)
    for vv in range(8):
        for cc in range(8):
            for uu in range(8):
                perm[vv * 64 + cc * 8 + uu] = cc * 64 + uu * 8 + vv
    wa_big = wa_big[:, perm]
    wb_big = _conv2d_toeplitz(w_conv1.astype(F32), h, w, pad=1, dil=1)
    x3 = x.reshape(b, 64, 512)

    ya, s_a, q_a = pl.pallas_call(
        functools.partial(_pass1_kernel, bb=bb1),
        grid=(cores, nt),
        in_specs=[
            pl.BlockSpec((bb1, 64, 512), lambda ci, i: (ci * nt + i, 0, 0)),
            pl.BlockSpec((512, 512), lambda ci, i: (0, 0)),
        ],
        out_specs=(
            pl.BlockSpec((512, tn), lambda ci, i: (0, ci * nt + i)),
            pl.BlockSpec((1, 512, 1), lambda ci, i: (ci, 0, 0)),
            pl.BlockSpec((1, 512, 1), lambda ci, i: (ci, 0, 0)),
        ),
        out_shape=(
            jax.ShapeDtypeStruct((512, n_a), jnp.bfloat16),
            jax.ShapeDtypeStruct((cores, 512, 1), F32),
            jax.ShapeDtypeStruct((cores, 512, 1), F32),
        ),
        compiler_params=pltpu.CompilerParams(
            dimension_semantics=("parallel", "arbitrary")),
        cost_estimate=pl.CostEstimate(
            flops=2 * 512 * 512 * n_a, transcendentals=0,
            bytes_accessed=8 * 512 * n_a),
    )(x3, wa_big)

    s_a = jnp.sum(s_a[:, :, 0], axis=0).reshape(c, u * v).sum(axis=1)
    q_a = jnp.sum(q_a[:, :, 0], axis=0).reshape(c, u * v).sum(axis=1)
    scale_a, shift_a = _bn_affine(s_a, q_a, u * v * n_a,
                                  g2.astype(F32), b2.astype(F32), eps)
    sa_rows = jnp.repeat(scale_a, h * w)[:, None]
    ta_rows = jnp.repeat(shift_a, h * w)[:, None]

    yb, s_b, q_b, gram = pl.pallas_call(
        _pass2_kernel,
        grid=(cores, nt),
        in_specs=[
            pl.BlockSpec((512, tn), lambda ci, i: (0, ci * nt + i)),
            pl.BlockSpec((512, 512), lambda ci, i: (0, 0)),
            pl.BlockSpec((512, 1), lambda ci, i: (0, 0)),
            pl.BlockSpec((512, 1), lambda ci, i: (0, 0)),
        ],
        out_specs=(
            pl.BlockSpec((512, tn), lambda ci, i: (0, ci * nt + i)),
            pl.BlockSpec((1, 512, 1), lambda ci, i: (ci, 0, 0)),
            pl.BlockSpec((1, 512, 1), lambda ci, i: (ci, 0, 0)),
            pl.BlockSpec((1, 512, 512), lambda ci, i: (ci, 0, 0)),
        ),
        out_shape=(
            jax.ShapeDtypeStruct((512, n_a), jnp.bfloat16),
            jax.ShapeDtypeStruct((cores, 512, 1), F32),
            jax.ShapeDtypeStruct((cores, 512, 1), F32),
            jax.ShapeDtypeStruct((cores, 512, 512), F32),
        ),
        compiler_params=pltpu.CompilerParams(
            dimension_semantics=("parallel", "arbitrary")),
        cost_estimate=pl.CostEstimate(
            flops=4 * 512 * 512 * n_a, transcendentals=0,
            bytes_accessed=8 * 512 * n_a),
    )(ya, wb_big, sa_rows, ta_rows)

    s_b = jnp.sum(s_b[:, :, 0], axis=0)
    q_b = jnp.sum(q_b[:, :, 0], axis=0)
    zsum = jnp.sum(gram, axis=0)
    hwmask = jnp.tile(jnp.eye(h * w, dtype=F32), (c, c))
    gram = (zsum * hwmask).reshape(c, h * w, c * h * w).sum(axis=1)
    gram = gram.reshape(c, c, h * w).sum(axis=2)
    s_bc = s_b.reshape(c, h * w).sum(axis=1)
    q_bc = q_b.reshape(c, h * w).sum(axis=1)
    scale_b, shift_b = _bn_affine(s_bc, q_bc, h * w * n_a,
                                  g1.astype(F32), b1.astype(F32), eps)

    co = w_proj.shape[0]
    wp2 = w_proj.reshape(co, c).astype(F32)
    wpp = wp2 * scale_b[None, :]
    cst = wp2 @ shift_b
    s_x = s_bc
    s3 = wpp @ s_x + n_c * cst
    q3 = (jnp.einsum("oc,cd,od->o", wpp, gram, wpp)
          + 2.0 * cst * (wpp @ s_x) + n_c * cst * cst)
    scale_c, shift_c = _bn_affine(s3, q3, n_c, gp.astype(F32),
                                  bp.astype(F32), eps)
    wf = scale_c[:, None] * wpp
    cf = scale_c * cst + shift_c

    bb3 = 16
    nt3 = b // (cores * bb3)
    wf_bd = jnp.kron(jnp.eye(bb3, dtype=F32), wf)
    cf_bd = jnp.tile(cf, bb3)[:, None]

    out2 = pl.pallas_call(
        functools.partial(_pass3_kernel, bb=bb3),
        grid=(cores, nt3),
        in_specs=[
            pl.BlockSpec((512, bb3 * 64), lambda ci, i: (0, ci * nt3 + i)),
            pl.BlockSpec((bb3 * co, bb3 * c), lambda ci, i: (0, 0)),
            pl.BlockSpec((bb3 * co, 1), lambda ci, i: (0, 0)),
        ],
        out_specs=pl.BlockSpec((bb3 * co, 4096),
                               lambda ci, i: (ci * nt3 + i, 0)),
        out_shape=jax.ShapeDtypeStruct((b * co, 4096), F32),
        compiler_params=pltpu.CompilerParams(
            dimension_semantics=("parallel", "arbitrary")),
        cost_estimate=pl.CostEstimate(
            flops=2 * co * c * 4096 * b, transcendentals=0,
            bytes_accessed=4 * (512 * n_a + b * co * 4096)),
    )(yb, wf_bd, cf_bd)

    return out2.reshape(b, co, u, v, h, w)


# speedup vs baseline: 4.7422x; 4.7422x over previous
import functools

import jax
import jax.numpy as jnp
import numpy as np
from jax.experimental import pallas as pl
from jax.experimental.pallas import tpu as pltpu

F32 = jnp.float32
BF16 = jnp.bfloat16


def _shift_masks(n, transpose_cols):
    del transpose_cols
    masks = []
    pi = np.arange(n)
    for a in range(3):
        for b in range(3):
            ra = (pi[:, None] == pi[None, :] + a - 1).astype(np.float32)
            rb = (pi[:, None] == pi[None, :] + b - 1).astype(np.float32)
            d = np.einsum("ip,jq->pqij", ra, rb).reshape(n * n, n * n)
            masks.append(d)
    return masks


def _toeplitz_rows_c_major(wk, masks, row_dim, col_perm_hw_c=False):
    co, ci, _, _ = wk.shape
    n2 = row_dim
    acc = None
    for k in range(9):
        a, b = divmod(k, 3)
        tap = wk[:, :, a, b].astype(F32)
        d = masks[k]
        if col_perm_hw_c:
            dt = np.repeat(np.tile(d, (co, 1)), ci, axis=1)
            ap = jnp.broadcast_to(tap[:, None, None, :], (co, n2, n2, ci))
            ap = ap.reshape(co * n2, n2 * ci)
        else:
            dt = np.tile(d, (co, ci))
            ap = jnp.broadcast_to(tap[:, None, :, None], (co, n2, ci, n2))
            ap = ap.reshape(co * n2, ci * n2)
        term = ap * jnp.asarray(dt)
        acc = term if acc is None else acc + term
    return acc


def _bn_affine(rowsum, rowssq, count, gamma, beta, eps):
    mean = rowsum / count
    var = rowssq / count - mean * mean
    inv_std = jax.lax.rsqrt(var + eps)
    scale = gamma * inv_std
    shift = beta - mean * scale
    return scale, shift


def _pass1_kernel(x_ref, w_ref, y_ref, sum_ref, ssq_ref, *, hwb, bsz):
    @pl.when(pl.program_id(1) == 0)
    def _():
        sum_ref[...] = jnp.zeros_like(sum_ref)
        ssq_ref[...] = jnp.zeros_like(ssq_ref)

    y = jnp.dot(w_ref[...], x_ref[...], preferred_element_type=F32)
    sum_ref[0] += jnp.sum(y, axis=1, keepdims=True)
    ssq_ref[0] += jnp.sum(y * y, axis=1, keepdims=True)
    t = y.reshape(8, 64, hwb, bsz).transpose(2, 0, 1, 3)
    y_ref[...] = t.reshape(hwb * 8, 64 * bsz).astype(BF16)


def _pass2_kernel(x_ref, w_ref, sc_ref, sh_ref, y_ref, sum_ref, ssq_ref,
                  gram_ref):
    x = jnp.maximum(x_ref[...].astype(F32) * sc_ref[...] + sh_ref[...], 0.0)
    y = jnp.dot(w_ref[...], x, preferred_element_type=F32)
    y_ref[...] = y.astype(BF16)

    @pl.when(pl.program_id(1) == 0)
    def _():
        sum_ref[...] = jnp.zeros_like(sum_ref)
        ssq_ref[...] = jnp.zeros_like(ssq_ref)
        gram_ref[...] = jnp.zeros_like(gram_ref)

    sum_ref[0] += jnp.sum(y, axis=1, keepdims=True)
    ssq_ref[0] += jnp.sum(y * y, axis=1, keepdims=True)
    g = jax.lax.dot_general(y, y, (((1,), (1,)), ((), ())),
                            preferred_element_type=F32)
    gram_ref[0] += g


def _pass3_kernel(x_ref, w_ref, c_ref, o_ref, *, uvb, bsz):
    x = x_ref[...].astype(F32)
    z = jnp.dot(w_ref[...], x, preferred_element_type=F32) + c_ref[...]
    z4 = z.reshape(16, 64, uvb, bsz).transpose(0, 2, 1, 3)
    o_ref[...] = z4.reshape(16, uvb * 64, bsz)


def kernel(x, w_conv2, g2, b2, w_conv1, g1, b1, w_proj, gp, bp):
    eps = 1e-5
    b, c, u, v, h, w = x.shape
    assert (c, u, v, h, w) == (8, 8, 8, 8, 8) and b % 128 == 0
    x = x.astype(F32)
    n_a = b * h * w
    n_c = n_a * 64
    co = w_proj.shape[0]

    cores = 2
    hwb = 8
    uvb = 8
    nt = 64 // (cores * hwb)

    masks = _shift_masks(8, False)
    wa_big = _toeplitz_rows_c_major(w_conv2, masks, 64)
    wb_big = _toeplitz_rows_c_major(w_conv1, masks, 64,
                                    col_perm_hw_c=True)

    xa = jnp.transpose(x, (1, 2, 3, 4, 5, 0)).reshape(512, 64 * b)

    ya, s_a, q_a = pl.pallas_call(
        functools.partial(_pass1_kernel, hwb=hwb, bsz=b),
        grid=(cores, nt),
        in_specs=[
            pl.BlockSpec((512, hwb * b), lambda ci, i: (0, ci * nt + i)),
            pl.BlockSpec((512, 512), lambda ci, i: (0, 0)),
        ],
        out_specs=(
            pl.BlockSpec((hwb * 8, 64 * b), lambda ci, i: (ci * nt + i, 0)),
            pl.BlockSpec((1, 512, 1), lambda ci, i: (ci, 0, 0)),
            pl.BlockSpec((1, 512, 1), lambda ci, i: (ci, 0, 0)),
        ),
        out_shape=(
            jax.ShapeDtypeStruct((512, 64 * b), BF16),
            jax.ShapeDtypeStruct((cores, 512, 1), F32),
            jax.ShapeDtypeStruct((cores, 512, 1), F32),
        ),
        compiler_params=pltpu.CompilerParams(
            dimension_semantics=("parallel", "arbitrary")),
        cost_estimate=pl.CostEstimate(
            flops=2 * 512 * 512 * n_a, transcendentals=0,
            bytes_accessed=6 * 512 * n_a),
    )(xa, wa_big)

    s_a = jnp.sum(s_a[:, :, 0], axis=0).reshape(c, u * v).sum(axis=1)
    q_a = jnp.sum(q_a[:, :, 0], axis=0).reshape(c, u * v).sum(axis=1)
    scale_a, shift_a = _bn_affine(s_a, q_a, u * v * n_a,
                                  g2.astype(F32), b2.astype(F32), eps)
    sa_rows = jnp.tile(scale_a, h * w)[:, None]
    ta_rows = jnp.tile(shift_a, h * w)[:, None]

    yb, s_b, q_b, gram = pl.pallas_call(
        _pass2_kernel,
        grid=(cores, nt),
        in_specs=[
            pl.BlockSpec((512, uvb * b), lambda ci, i: (0, ci * nt + i)),
            pl.BlockSpec((512, 512), lambda ci, i: (0, 0)),
            pl.BlockSpec((512, 1), lambda ci, i: (0, 0)),
            pl.BlockSpec((512, 1), lambda ci, i: (0, 0)),
        ],
        out_specs=(
            pl.BlockSpec((512, uvb * b), lambda ci, i: (0, ci * nt + i)),
            pl.BlockSpec((1, 512, 1), lambda ci, i: (ci, 0, 0)),
            pl.BlockSpec((1, 512, 1), lambda ci, i: (ci, 0, 0)),
            pl.BlockSpec((1, 512, 512), lambda ci, i: (ci, 0, 0)),
        ),
        out_shape=(
            jax.ShapeDtypeStruct((512, 64 * b), BF16),
            jax.ShapeDtypeStruct((cores, 512, 1), F32),
            jax.ShapeDtypeStruct((cores, 512, 1), F32),
            jax.ShapeDtypeStruct((cores, 512, 512), F32),
        ),
        compiler_params=pltpu.CompilerParams(
            dimension_semantics=("parallel", "arbitrary")),
        cost_estimate=pl.CostEstimate(
            flops=3 * 512 * 512 * n_a, transcendentals=0,
            bytes_accessed=4 * 512 * n_a),
    )(ya, wb_big, sa_rows, ta_rows)

    s_b = jnp.sum(s_b[:, :, 0], axis=0)
    q_b = jnp.sum(q_b[:, :, 0], axis=0)
    s_bc = s_b.reshape(c, h * w).sum(axis=1)
    q_bc = q_b.reshape(c, h * w).sum(axis=1)
    scale_b, shift_b = _bn_affine(s_bc, q_bc, h * w * n_a,
                                  g1.astype(F32), b1.astype(F32), eps)

    zsum = jnp.sum(gram, axis=0)
    hwmask = jnp.tile(jnp.eye(h * w, dtype=F32), (c, c))
    gch = (zsum * hwmask).reshape(c, h * w, c * h * w).sum(axis=1)
    gch = gch.reshape(c, c, h * w).sum(axis=2)

    wp2 = w_proj.reshape(co, c).astype(F32)
    wpp = wp2 * scale_b[None, :]
    cst = wp2 @ shift_b
    s3 = wpp @ s_bc + n_c * cst
    q3 = (jnp.einsum("oc,cd,od->o", wpp, gch, wpp)
          + 2.0 * cst * (wpp @ s_bc) + n_c * cst * cst)
    scale_c, shift_c = _bn_affine(s3, q3, n_c, gp.astype(F32),
                                  bp.astype(F32), eps)
    wf = scale_c[:, None] * wpp
    cf = scale_c * cst + shift_c

    eye_t = np.tile(np.eye(64, dtype=np.float32), (co, c))
    wf_kron = (jnp.broadcast_to(wf[:, None, :, None], (co, 64, c, 64))
               .reshape(co * 64, c * 64) * jnp.asarray(eye_t))
    cf_rows = jnp.broadcast_to(cf[:, None], (co, 64)).reshape(co * 64)[:, None]

    out3 = pl.pallas_call(
        functools.partial(_pass3_kernel, uvb=uvb, bsz=b),
        grid=(cores, nt),
        in_specs=[
            pl.BlockSpec((512, uvb * b), lambda ci, i: (0, ci * nt + i)),
            pl.BlockSpec((co * 64, 512), lambda ci, i: (0, 0)),
            pl.BlockSpec((co * 64, 1), lambda ci, i: (0, 0)),
        ],
        out_specs=pl.BlockSpec((co, uvb * 64, b),
                               lambda ci, i: (0, ci * nt + i, 0)),
        out_shape=jax.ShapeDtypeStruct((co, 4096, b), F32),
        compiler_params=pltpu.CompilerParams(
            dimension_semantics=("parallel", "arbitrary")),
        cost_estimate=pl.CostEstimate(
            flops=2 * co * 64 * 512 * n_a, transcendentals=0,
            bytes_accessed=2 * 512 * n_a + 4 * co * 64 * n_a),
    )(yb, wf_kron, cf_rows)

    return jnp.transpose(out3.reshape(co, u, v, h, w, b),
                         (5, 0, 1, 2, 3, 4))


# speedup vs baseline: 5.7074x; 1.2035x over previous
import functools

import jax
import jax.numpy as jnp
import numpy as np
from jax.experimental import pallas as pl
from jax.experimental.pallas import tpu as pltpu

F32 = jnp.float32
BF16 = jnp.bfloat16


def _shift_masks(n, transpose_cols):
    del transpose_cols
    masks = []
    pi = np.arange(n)
    for a in range(3):
        for b in range(3):
            ra = (pi[:, None] == pi[None, :] + a - 1).astype(np.float32)
            rb = (pi[:, None] == pi[None, :] + b - 1).astype(np.float32)
            d = np.einsum("ip,jq->pqij", ra, rb).reshape(n * n, n * n)
            masks.append(d)
    return masks


def _toeplitz_rows_c_major(wk, masks, row_dim, col_perm_hw_c=False):
    co, ci, _, _ = wk.shape
    n2 = row_dim
    acc = None
    for k in range(9):
        a, b = divmod(k, 3)
        tap = wk[:, :, a, b].astype(F32)
        d = masks[k]
        if col_perm_hw_c:
            dt = np.repeat(np.tile(d, (co, 1)), ci, axis=1)
            ap = jnp.broadcast_to(tap[:, None, None, :], (co, n2, n2, ci))
            ap = ap.reshape(co * n2, n2 * ci)
        else:
            dt = np.tile(d, (co, ci))
            ap = jnp.broadcast_to(tap[:, None, :, None], (co, n2, ci, n2))
            ap = ap.reshape(co * n2, ci * n2)
        term = ap * jnp.asarray(dt)
        acc = term if acc is None else acc + term
    return acc


def _bn_affine(rowsum, rowssq, count, gamma, beta, eps):
    mean = rowsum / count
    var = rowssq / count - mean * mean
    inv_std = jax.lax.rsqrt(var + eps)
    scale = gamma * inv_std
    shift = beta - mean * scale
    return scale, shift


def _pass1_kernel(x_ref, w_ref, y_ref, sum_ref, ssq_ref, *, hwb, bsz):
    @pl.when(pl.program_id(1) == 0)
    def _():
        sum_ref[...] = jnp.zeros_like(sum_ref)
        ssq_ref[...] = jnp.zeros_like(ssq_ref)

    xp = jnp.concatenate([x_ref[:, k, :] for k in range(hwb)], axis=1)
    y = jnp.dot(w_ref[...], xp, preferred_element_type=F32)
    sum_ref[0] += jnp.sum(y, axis=1, keepdims=True)
    ssq_ref[0] += jnp.sum(y * y, axis=1, keepdims=True)
    t = y.reshape(8, 64, hwb, bsz).transpose(2, 0, 1, 3)
    y_ref[...] = t.reshape(hwb * 8, 64 * bsz).astype(BF16)


def _pass2_kernel(x_ref, w_ref, sc_ref, sh_ref, y_ref, sum_ref, ssq_ref,
                  gram_ref):
    x = jnp.maximum(x_ref[...].astype(F32) * sc_ref[...] + sh_ref[...], 0.0)
    y = jnp.dot(w_ref[...], x, preferred_element_type=F32)
    y_ref[...] = y.astype(BF16)

    @pl.when(pl.program_id(1) == 0)
    def _():
        sum_ref[...] = jnp.zeros_like(sum_ref)
        ssq_ref[...] = jnp.zeros_like(ssq_ref)
        gram_ref[...] = jnp.zeros_like(gram_ref)

    sum_ref[0] += jnp.sum(y, axis=1, keepdims=True)
    ssq_ref[0] += jnp.sum(y * y, axis=1, keepdims=True)
    g = jax.lax.dot_general(y, y, (((1,), (1,)), ((), ())),
                            preferred_element_type=F32)
    gram_ref[0] += g


def _pass3_kernel(x_ref, w_ref, c_ref, o_ref, *, uvb, bsz):
    x = x_ref[...].astype(F32)
    z = jnp.dot(w_ref[...], x, preferred_element_type=F32) + c_ref[...]
    z4 = z.reshape(16, 64, uvb, bsz).transpose(0, 2, 1, 3)
    o_ref[...] = z4.reshape(16, uvb * 64, bsz)


def kernel(x, w_conv2, g2, b2, w_conv1, g1, b1, w_proj, gp, bp):
    eps = 1e-5
    b, c, u, v, h, w = x.shape
    assert (c, u, v, h, w) == (8, 8, 8, 8, 8) and b % 128 == 0
    x = x.astype(F32)
    n_a = b * h * w
    n_c = n_a * 64
    co = w_proj.shape[0]

    cores = 2
    hwb = 8
    uvb = 8
    nt = 64 // (cores * hwb)

    masks = _shift_masks(8, False)
    wa_big = _toeplitz_rows_c_major(w_conv2, masks, 64)
    wb_big = _toeplitz_rows_c_major(w_conv1, masks, 64,
                                    col_perm_hw_c=True)

    xa = jnp.transpose(x, (1, 2, 3, 4, 5, 0)).reshape(512, 64, b)

    ya, s_a, q_a = pl.pallas_call(
        functools.partial(_pass1_kernel, hwb=hwb, bsz=b),
        grid=(cores, nt),
        in_specs=[
            pl.BlockSpec((512, hwb, b), lambda ci, i: (0, ci * nt + i, 0)),
            pl.BlockSpec((512, 512), lambda ci, i: (0, 0)),
        ],
        out_specs=(
            pl.BlockSpec((hwb * 8, 64 * b), lambda ci, i: (ci * nt + i, 0)),
            pl.BlockSpec((1, 512, 1), lambda ci, i: (ci, 0, 0)),
            pl.BlockSpec((1, 512, 1), lambda ci, i: (ci, 0, 0)),
        ),
        out_shape=(
            jax.ShapeDtypeStruct((512, 64 * b), BF16),
            jax.ShapeDtypeStruct((cores, 512, 1), F32),
            jax.ShapeDtypeStruct((cores, 512, 1), F32),
        ),
        compiler_params=pltpu.CompilerParams(
            dimension_semantics=("parallel", "arbitrary")),
        cost_estimate=pl.CostEstimate(
            flops=2 * 512 * 512 * n_a, transcendentals=0,
            bytes_accessed=6 * 512 * n_a),
    )(xa, wa_big)

    s_a = jnp.sum(s_a[:, :, 0], axis=0).reshape(c, u * v).sum(axis=1)
    q_a = jnp.sum(q_a[:, :, 0], axis=0).reshape(c, u * v).sum(axis=1)
    scale_a, shift_a = _bn_affine(s_a, q_a, u * v * n_a,
                                  g2.astype(F32), b2.astype(F32), eps)
    sa_rows = jnp.tile(scale_a, h * w)[:, None]
    ta_rows = jnp.tile(shift_a, h * w)[:, None]

    yb, s_b, q_b, gram = pl.pallas_call(
        _pass2_kernel,
        grid=(cores, nt),
        in_specs=[
            pl.BlockSpec((512, uvb * b), lambda ci, i: (0, ci * nt + i)),
            pl.BlockSpec((512, 512), lambda ci, i: (0, 0)),
            pl.BlockSpec((512, 1), lambda ci, i: (0, 0)),
            pl.BlockSpec((512, 1), lambda ci, i: (0, 0)),
        ],
        out_specs=(
            pl.BlockSpec((512, uvb * b), lambda ci, i: (0, ci * nt + i)),
            pl.BlockSpec((1, 512, 1), lambda ci, i: (ci, 0, 0)),
            pl.BlockSpec((1, 512, 1), lambda ci, i: (ci, 0, 0)),
            pl.BlockSpec((1, 512, 512), lambda ci, i: (ci, 0, 0)),
        ),
        out_shape=(
            jax.ShapeDtypeStruct((512, 64 * b), BF16),
            jax.ShapeDtypeStruct((cores, 512, 1), F32),
            jax.ShapeDtypeStruct((cores, 512, 1), F32),
            jax.ShapeDtypeStruct((cores, 512, 512), F32),
        ),
        compiler_params=pltpu.CompilerParams(
            dimension_semantics=("parallel", "arbitrary")),
        cost_estimate=pl.CostEstimate(
            flops=3 * 512 * 512 * n_a, transcendentals=0,
            bytes_accessed=4 * 512 * n_a),
    )(ya, wb_big, sa_rows, ta_rows)

    s_b = jnp.sum(s_b[:, :, 0], axis=0)
    q_b = jnp.sum(q_b[:, :, 0], axis=0)
    s_bc = s_b.reshape(c, h * w).sum(axis=1)
    q_bc = q_b.reshape(c, h * w).sum(axis=1)
    scale_b, shift_b = _bn_affine(s_bc, q_bc, h * w * n_a,
                                  g1.astype(F32), b1.astype(F32), eps)

    zsum = jnp.sum(gram, axis=0)
    hwmask = jnp.tile(jnp.eye(h * w, dtype=F32), (c, c))
    gch = (zsum * hwmask).reshape(c, h * w, c * h * w).sum(axis=1)
    gch = gch.reshape(c, c, h * w).sum(axis=2)

    wp2 = w_proj.reshape(co, c).astype(F32)
    wpp = wp2 * scale_b[None, :]
    cst = wp2 @ shift_b
    s3 = wpp @ s_bc + n_c * cst
    q3 = (jnp.einsum("oc,cd,od->o", wpp, gch, wpp)
          + 2.0 * cst * (wpp @ s_bc) + n_c * cst * cst)
    scale_c, shift_c = _bn_affine(s3, q3, n_c, gp.astype(F32),
                                  bp.astype(F32), eps)
    wf = scale_c[:, None] * wpp
    cf = scale_c * cst + shift_c

    eye_t = np.tile(np.eye(64, dtype=np.float32), (co, c))
    wf_kron = (jnp.broadcast_to(wf[:, None, :, None], (co, 64, c, 64))
               .reshape(co * 64, c * 64) * jnp.asarray(eye_t))
    cf_rows = jnp.broadcast_to(cf[:, None], (co, 64)).reshape(co * 64)[:, None]

    out3 = pl.pallas_call(
        functools.partial(_pass3_kernel, uvb=uvb, bsz=b),
        grid=(cores, nt),
        in_specs=[
            pl.BlockSpec((512, uvb * b), lambda ci, i: (0, ci * nt + i)),
            pl.BlockSpec((co * 64, 512), lambda ci, i: (0, 0)),
            pl.BlockSpec((co * 64, 1), lambda ci, i: (0, 0)),
        ],
        out_specs=pl.BlockSpec((co, uvb * 64, b),
                               lambda ci, i: (0, ci * nt + i, 0)),
        out_shape=jax.ShapeDtypeStruct((co, 4096, b), F32),
        compiler_params=pltpu.CompilerParams(
            dimension_semantics=("parallel", "arbitrary")),
        cost_estimate=pl.CostEstimate(
            flops=2 * co * 64 * 512 * n_a, transcendentals=0,
            bytes_accessed=2 * 512 * n_a + 4 * co * 64 * n_a),
    )(yb, wf_kron, cf_rows)

    return jnp.transpose(out3.reshape(co, u, v, h, w, b),
                         (5, 0, 1, 2, 3, 4))
